# baseline (speedup 1.0000x reference)
import jax
import jax.numpy as jnp
from jax.experimental import pallas as pl

_TM = 512


def _router_kernel(x_ref, w_ref, probs_ref, logits_ref):
    logits = jnp.dot(x_ref[...], w_ref[...], preferred_element_type=jnp.float32)
    logits_ref[...] = logits
    m = jnp.max(logits, axis=-1, keepdims=True)
    e = jnp.exp(logits - m)
    s = jnp.sum(e, axis=-1, keepdims=True)
    probs_ref[...] = e / s


def kernel(token_inputs, W, expert_capacity):
    g, t, h = token_inputs.shape
    e = W.shape[1]
    n = g * t
    x = token_inputs.reshape(n, h)
    probs, logits = pl.pallas_call(
        _router_kernel,
        grid=(n // _TM,),
        in_specs=[
            pl.BlockSpec((_TM, h), lambda i: (i, 0)),
            pl.BlockSpec((h, e), lambda i: (0, 0)),
        ],
        out_specs=[
            pl.BlockSpec((_TM, e), lambda i: (i, 0)),
            pl.BlockSpec((_TM, e), lambda i: (i, 0)),
        ],
        out_shape=[
            jax.ShapeDtypeStruct((n, e), jnp.float32),
            jax.ShapeDtypeStruct((n, e), jnp.float32),
        ],
    )(x, W)
    lse = jax.scipy.special.logsumexp(logits, axis=-1)
    z_loss = jnp.sum(lse**2) / n
    return probs.reshape(g, t, e), logits.reshape(g, t, e), z_loss


# speedup vs baseline: 1.0760x; 1.0760x over previous
import jax
import jax.numpy as jnp
from jax.experimental import pallas as pl

_TM = 512


def _router_kernel(x_ref, w_ref, probs_ref, logits_ref, z_ref):
    gi = pl.program_id(0)
    ti = pl.program_id(1)
    logits = jnp.dot(x_ref[0], w_ref[...], preferred_element_type=jnp.float32)
    logits_ref[0] = logits
    m = jnp.max(logits, axis=-1, keepdims=True)
    e = jnp.exp(logits - m)
    s = jnp.sum(e, axis=-1, keepdims=True)
    probs_ref[0] = e / s
    lse = m + jnp.log(s)
    part = jnp.sum(lse * lse, keepdims=True)

    @pl.when((gi == 0) & (ti == 0))
    def _init():
        z_ref[...] = part

    @pl.when((gi != 0) | (ti != 0))
    def _acc():
        z_ref[...] += part


def kernel(token_inputs, W, expert_capacity):
    g, t, h = token_inputs.shape
    e = W.shape[1]
    probs, logits, z = pl.pallas_call(
        _router_kernel,
        grid=(g, t // _TM),
        in_specs=[
            pl.BlockSpec((1, _TM, h), lambda gi, ti: (gi, ti, 0)),
            pl.BlockSpec((h, e), lambda gi, ti: (0, 0)),
        ],
        out_specs=[
            pl.BlockSpec((1, _TM, e), lambda gi, ti: (gi, ti, 0)),
            pl.BlockSpec((1, _TM, e), lambda gi, ti: (gi, ti, 0)),
            pl.BlockSpec((1, 1), lambda gi, ti: (0, 0)),
        ],
        out_shape=[
            jax.ShapeDtypeStruct((g, t, e), jnp.float32),
            jax.ShapeDtypeStruct((g, t, e), jnp.float32),
            jax.ShapeDtypeStruct((1, 1), jnp.float32),
        ],
    )(token_inputs, W)
    z_loss = z[0, 0] / (g * t)
    return probs, logits, z_loss
